# Initial kernel scaffold; baseline (speedup 1.0000x reference)
import functools

import jax
import jax.numpy as jnp
from jax import lax
from jax.experimental import pallas as pl
from jax.experimental.pallas import tpu as pltpu
from jax.experimental.pallas import tpu_sc as plsc

_N = 10000
_E = 320000
_F = 128
_H = 30
_HP = 32
_NG = 64

_NC, _NS = 2, 16
_NW = _NC * _NS
_BLK = 128
_EBT = 79
_NBLKS = _NW * _EBT
_EPAD = _NBLKS * _BLK
_NPAD = 10016
_RPT = _NPAD // _NS


def _sc_segsum(y, srcb, dstb, zeros):
    mesh = plsc.VectorSubcoreMesh(
        core_axis_name="c", subcore_axis_name="s",
        num_cores=_NC, num_subcores=_NS)

    @functools.partial(
        pl.kernel,
        out_type=jax.ShapeDtypeStruct((_NC, _NPAD, _HP), jnp.float32),
        mesh=mesh,
        scratch_types=[
            pltpu.VMEM((_EBT, _BLK), jnp.int32),
            pltpu.VMEM((_EBT, _BLK), jnp.int32),
            pltpu.VMEM((_BLK, _HP), jnp.float32),
            pltpu.VMEM((_RPT, _HP), jnp.float32),
            pltpu.VMEM_SHARED((_NPAD, _HP), jnp.float32),
            pltpu.SemaphoreType.DMA,
        ],
    )
    def k(y_hbm, srcb_hbm, dstb_hbm, zeros_hbm, out_hbm,
          src_v, dst_v, rows_v, stage_v, acc_sh, sem):
        c = lax.axis_index("c")
        s = lax.axis_index("s")
        wid = s * _NC + c
        pltpu.sync_copy(zeros_hbm.at[pl.ds(s * _RPT, _RPT)], stage_v)
        pltpu.sync_copy(stage_v, acc_sh.at[pl.ds(s * _RPT, _RPT)])
        base = wid * _EBT
        pltpu.sync_copy(srcb_hbm.at[pl.ds(base, _EBT)], src_v)
        pltpu.sync_copy(dstb_hbm.at[pl.ds(base, _EBT)], dst_v)
        plsc.subcore_barrier()

        def body(j, carry):
            pltpu.async_copy(y_hbm.at[src_v.at[j]], rows_v, sem).wait()
            pltpu.sync_copy(rows_v, acc_sh.at[dst_v.at[j]], add=True)
            return carry

        lax.fori_loop(0, _EBT, body, 0)
        plsc.subcore_barrier()
        pltpu.sync_copy(acc_sh.at[pl.ds(s * _RPT, _RPT)], stage_v)
        pltpu.sync_copy(stage_v, out_hbm.at[c, pl.ds(s * _RPT, _RPT)])

    return k(y, srcb, dstb, zeros)


def _tc_dense_in(x, Wlp, Wrp):
    def body(x_ref, wl_ref, wr_ref, y_ref, r_ref):
        xv = x_ref[...]
        y = jnp.dot(xv, wl_ref[...], preferred_element_type=jnp.float32)
        lane = lax.broadcasted_iota(jnp.int32, (_N, _HP), 1)
        y_ref[...] = jnp.where(lane == _HP - 1, 1.0, y)
        r_ref[...] = jnp.dot(xv, wr_ref[...], preferred_element_type=jnp.float32)

    return pl.pallas_call(
        body,
        out_shape=(jax.ShapeDtypeStruct((_N, _HP), jnp.float32),
                   jax.ShapeDtypeStruct((_N, _HP), jnp.float32)),
    )(x, Wlp, Wrp)


def _combine_bn(p_ref, r_ref, bl_ref, g_ref, be_ref):
    agg = p_ref[0, :_N, :] + p_ref[1, :_N, :]
    deg = jnp.maximum(agg[:, _HP - 1:_HP], 1.0)
    h = jnp.maximum(agg / deg + bl_ref[...] + r_ref[...], 0.0)
    mu = jnp.mean(h, axis=0, keepdims=True)
    var = jnp.mean((h - mu) * (h - mu), axis=0, keepdims=True)
    return (h - mu) * lax.rsqrt(var + 1e-5) * g_ref[...] + be_ref[...]


def _tc_mid(parts, r1, bl1p, g1p, b1p, Wl2p, Wr2p):
    def body(p_ref, r_ref, bl_ref, g_ref, be_ref, wl_ref, wr_ref,
             y2_ref, r2_ref):
        hn = _combine_bn(p_ref, r_ref, bl_ref, g_ref, be_ref)
        y2 = jnp.dot(hn, wl_ref[...], preferred_element_type=jnp.float32)
        lane = lax.broadcasted_iota(jnp.int32, (_N, _HP), 1)
        y2_ref[...] = jnp.where(lane == _HP - 1, 1.0, y2)
        r2_ref[...] = jnp.dot(hn, wr_ref[...], preferred_element_type=jnp.float32)

    return pl.pallas_call(
        body,
        out_shape=(jax.ShapeDtypeStruct((_N, _HP), jnp.float32),
                   jax.ShapeDtypeStruct((_N, _HP), jnp.float32)),
    )(parts, r1, bl1p, g1p, b1p, Wl2p, Wr2p)


def _tc_final(parts, r2, bl2p, g2p, b2p, bt2d, W1a, W1b, bln1, W2p, bln2):
    def body(p_ref, r_ref, bl_ref, g_ref, be_ref, bt_ref,
             w1a_ref, w1b_ref, bn1_ref, w2_ref, bn2_ref, out_ref):
        hn = _combine_bn(p_ref, r_ref, bl_ref, g_ref, be_ref)
        bt = bt_ref[...]
        lane = lax.broadcasted_iota(jnp.int32, (_N, _HP), 1)
        hs = jnp.where(lane == _HP - 1, 1.0, hn)
        oh = (bt == lax.broadcasted_iota(jnp.int32, (_N, _NG), 1)
              ).astype(jnp.float32)
        ssum = lax.dot_general(oh, hs, (((0,), (0,)), ((), ())),
                               preferred_element_type=jnp.float32)
        cnt = jnp.maximum(ssum[:, _HP - 1:_HP], 1.0)
        x2 = ssum / cnt
        neg = jnp.float32(-jnp.inf)
        rows = [jnp.max(jnp.where(bt == g, hn, neg), axis=0, keepdims=True)
                for g in range(_NG)]
        x1 = jnp.concatenate(rows, axis=0)
        lane2 = lax.broadcasted_iota(jnp.int32, (_NG, _HP), 1)
        x1 = jnp.where(lane2 < _H, x1, 0.0)
        x2 = jnp.where(lane2 < _H, x2, 0.0)
        z = (jnp.dot(x1, w1a_ref[...], preferred_element_type=jnp.float32)
             + jnp.dot(x2, w1b_ref[...], preferred_element_type=jnp.float32)
             + bn1_ref[...])
        z = jnp.maximum(z, 0.0)
        o = jnp.dot(z, w2_ref[...], preferred_element_type=jnp.float32) \
            + bn2_ref[...]
        out_ref[...] = 1.0 / (1.0 + jnp.exp(-o))

    return pl.pallas_call(
        body,
        out_shape=jax.ShapeDtypeStruct((_NG, 8), jnp.float32),
    )(parts, r2, bl2p, g2p, b2p, bt2d, W1a, W1b, bln1, W2p, bln2)


def kernel(x, edge_index, batch, Wl1, bl1, Wr1, gamma1, beta1,
           Wl2, bl2, Wr2, gamma2, beta2, Wlin1, blin1, Wlin2, blin2):
    f32 = jnp.float32
    Wl1p = jnp.zeros((_F, _HP), f32).at[:, :_H].set(Wl1)
    Wr1p = jnp.zeros((_F, _HP), f32).at[:, :_H].set(Wr1)
    Wl2p = jnp.zeros((_HP, _HP), f32).at[:_H, :_H].set(Wl2)
    Wr2p = jnp.zeros((_HP, _HP), f32).at[:_H, :_H].set(Wr2)
    bl1p = jnp.zeros((1, _HP), f32).at[0, :_H].set(bl1)
    bl2p = jnp.zeros((1, _HP), f32).at[0, :_H].set(bl2)
    g1p = jnp.zeros((1, _HP), f32).at[0, :_H].set(gamma1)
    g2p = jnp.zeros((1, _HP), f32).at[0, :_H].set(gamma2)
    b1p = jnp.zeros((1, _HP), f32).at[0, :_H].set(beta1)
    b2p = jnp.zeros((1, _HP), f32).at[0, :_H].set(beta2)
    W1a = jnp.zeros((_HP, 16), f32).at[:_H, :10].set(Wlin1[:_H])
    W1b = jnp.zeros((_HP, 16), f32).at[:_H, :10].set(Wlin1[_H:])
    bln1 = jnp.zeros((1, 16), f32).at[0, :10].set(blin1)
    W2p = jnp.zeros((16, 8), f32).at[:10, 0].set(Wlin2[:, 0])
    bln2 = jnp.zeros((1, 8), f32).at[0, 0].set(blin2[0])

    src = edge_index[0]
    dst = edge_index[1]
    pad = _EPAD - _E
    srcb = jnp.concatenate(
        [src, jnp.zeros((pad,), jnp.int32)]).reshape(_NBLKS, _BLK)
    dstb = jnp.concatenate(
        [dst, jnp.full((pad,), _N, jnp.int32)]).reshape(_NBLKS, _BLK)
    zeros = jnp.zeros((_NPAD, _HP), f32)
    bt2d = batch.reshape(_N, 1)

    y1, r1 = _tc_dense_in(x, Wl1p, Wr1p)
    parts1 = _sc_segsum(y1, srcb, dstb, zeros)
    y2, r2 = _tc_mid(parts1, r1, bl1p, g1p, b1p, Wl2p, Wr2p)
    parts2 = _sc_segsum(y2, srcb, dstb, zeros)
    out = _tc_final(parts2, r2, bl2p, g2p, b2p, bt2d, W1a, W1b, bln1, W2p, bln2)
    return out[:, :1]


# speedup vs baseline: 6.5602x; 6.5602x over previous
import functools

import jax
import jax.numpy as jnp
from jax import lax
from jax.experimental import pallas as pl
from jax.experimental.pallas import tpu as pltpu
from jax.experimental.pallas import tpu_sc as plsc

_N = 10000
_E = 320000
_F = 128
_H = 30
_HP = 32
_NG = 64

_NC, _NS = 2, 16
_NW = _NC * _NS
_BLK = 128
_EBT = 80
_NBLKS = _NW * _EBT
_EPAD = _NBLKS * _BLK
_NPAD = 10112
_RPT = _NPAD // _NS

_sc_mesh = plsc.VectorSubcoreMesh(
    core_axis_name="c", subcore_axis_name="s",
    num_cores=_NC, num_subcores=_NS)


def _chunks(total, step):
    out = []
    off = 0
    while off < total:
        out.append((off, min(step, total - off)))
        off += step
    return out


@functools.partial(
    pl.kernel,
    out_type=jax.ShapeDtypeStruct((_EBT, _NW * _BLK, _HP), jnp.float32),
    mesh=_sc_mesh,
    scratch_types=[
        pltpu.VMEM((_EBT, _BLK), jnp.int32),
        pltpu.VMEM((_BLK, _HP), jnp.float32),
        pltpu.VMEM_SHARED((_NPAD, _HP), jnp.float32),
        pltpu.SemaphoreType.DMA,
    ],
)
def _sc_gather(y_hbm, srcb_hbm, msgs_hbm, src_v, rows_v, y_sh, sem):
    c = lax.axis_index("c")
    s = lax.axis_index("s")
    wid = s * _NC + c
    for off, nrows in _chunks(_RPT, _BLK):
        pltpu.sync_copy(y_hbm.at[pl.ds(s * _RPT + off, nrows)],
                        rows_v.at[pl.ds(0, nrows)])
        pltpu.sync_copy(rows_v.at[pl.ds(0, nrows)],
                        y_sh.at[pl.ds(s * _RPT + off, nrows)])
    base = wid * _EBT
    pltpu.sync_copy(srcb_hbm.at[pl.ds(base, _EBT)], src_v)
    plsc.subcore_barrier()

    @pl.loop(0, _EBT)
    def _(j):
        pltpu.async_copy(y_sh.at[src_v.at[j]], rows_v, sem).wait()
        pltpu.sync_copy(rows_v, msgs_hbm.at[j, pl.ds(wid * _BLK, _BLK)])


@functools.partial(
    pl.kernel,
    out_type=jax.ShapeDtypeStruct((_NC, _NPAD, _HP), jnp.float32),
    mesh=_sc_mesh,
    scratch_types=[
        pltpu.VMEM((_EBT, _BLK), jnp.int32),
        pltpu.VMEM((_BLK, _HP), jnp.float32),
        pltpu.VMEM_SHARED((_NPAD, _HP), jnp.float32),
        pltpu.SemaphoreType.DMA,
    ],
)
def _sc_scatter(msgs_hbm, dstb_hbm, zeros_hbm, out_hbm,
                dst_v, rows_v, acc_sh, sem):
    c = lax.axis_index("c")
    s = lax.axis_index("s")
    wid = s * _NC + c
    for off, nrows in _chunks(_RPT, _BLK):
        pltpu.sync_copy(zeros_hbm.at[pl.ds(s * _RPT + off, nrows)],
                        rows_v.at[pl.ds(0, nrows)])
        pltpu.sync_copy(rows_v.at[pl.ds(0, nrows)],
                        acc_sh.at[pl.ds(s * _RPT + off, nrows)])
    base = wid * _EBT
    pltpu.sync_copy(dstb_hbm.at[pl.ds(base, _EBT)], dst_v)
    plsc.subcore_barrier()

    @pl.loop(0, _EBT)
    def _(j):
        pltpu.sync_copy(msgs_hbm.at[j, pl.ds(wid * _BLK, _BLK)], rows_v)
        pltpu.sync_copy(rows_v, acc_sh.at[dst_v.at[j]], add=True)
    plsc.subcore_barrier()
    for off, nrows in _chunks(_RPT, _BLK):
        pltpu.sync_copy(acc_sh.at[pl.ds(s * _RPT + off, nrows)],
                        rows_v.at[pl.ds(0, nrows)])
        pltpu.sync_copy(rows_v.at[pl.ds(0, nrows)],
                        out_hbm.at[c, pl.ds(s * _RPT + off, nrows)])


def _sc_segsum(y, srcb, dstb, zeros):
    msgs = _sc_gather(y, srcb)
    return _sc_scatter(msgs, dstb, zeros)


def _tc_dense_in(x, Wlp, Wrp):
    def body(x_ref, wl_ref, wr_ref, y_ref, r_ref):
        xv = x_ref[...]
        y = jnp.dot(xv, wl_ref[...], preferred_element_type=jnp.float32)
        lane = lax.broadcasted_iota(jnp.int32, (_N, _HP), 1)
        y_ref[pl.ds(0, _N), :] = jnp.where(lane == _HP - 1, 1.0, y)
        y_ref[pl.ds(_N, _NPAD - _N), :] = jnp.zeros(
            (_NPAD - _N, _HP), jnp.float32)
        r_ref[...] = jnp.dot(xv, wr_ref[...], preferred_element_type=jnp.float32)

    return pl.pallas_call(
        body,
        out_shape=(jax.ShapeDtypeStruct((_NPAD, _HP), jnp.float32),
                   jax.ShapeDtypeStruct((_N, _HP), jnp.float32)),
    )(x, Wlp, Wrp)


def _combine_bn(p_ref, r_ref, bl_ref, g_ref, be_ref):
    agg = p_ref[0, :_N, :] + p_ref[1, :_N, :]
    deg = jnp.maximum(agg[:, _HP - 1:_HP], 1.0)
    h = jnp.maximum(agg / deg + bl_ref[...] + r_ref[...], 0.0)
    mu = jnp.mean(h, axis=0, keepdims=True)
    var = jnp.mean((h - mu) * (h - mu), axis=0, keepdims=True)
    return (h - mu) * lax.rsqrt(var + 1e-5) * g_ref[...] + be_ref[...]


def _tc_mid(parts, r1, bl1p, g1p, b1p, Wl2p, Wr2p):
    def body(p_ref, r_ref, bl_ref, g_ref, be_ref, wl_ref, wr_ref,
             y2_ref, r2_ref):
        hn = _combine_bn(p_ref, r_ref, bl_ref, g_ref, be_ref)
        y2 = jnp.dot(hn, wl_ref[...], preferred_element_type=jnp.float32)
        lane = lax.broadcasted_iota(jnp.int32, (_N, _HP), 1)
        y2_ref[pl.ds(0, _N), :] = jnp.where(lane == _HP - 1, 1.0, y2)
        y2_ref[pl.ds(_N, _NPAD - _N), :] = jnp.zeros(
            (_NPAD - _N, _HP), jnp.float32)
        r2_ref[...] = jnp.dot(hn, wr_ref[...], preferred_element_type=jnp.float32)

    return pl.pallas_call(
        body,
        out_shape=(jax.ShapeDtypeStruct((_NPAD, _HP), jnp.float32),
                   jax.ShapeDtypeStruct((_N, _HP), jnp.float32)),
    )(parts, r1, bl1p, g1p, b1p, Wl2p, Wr2p)


def _tc_final(parts, r2, bl2p, g2p, b2p, bt2d, W1a, W1b, bln1, W2p, bln2):
    def body(p_ref, r_ref, bl_ref, g_ref, be_ref, bt_ref,
             w1a_ref, w1b_ref, bn1_ref, w2_ref, bn2_ref, out_ref):
        hn = _combine_bn(p_ref, r_ref, bl_ref, g_ref, be_ref)
        bt = bt_ref[...]
        lane = lax.broadcasted_iota(jnp.int32, (_N, _HP), 1)
        hs = jnp.where(lane == _HP - 1, 1.0, hn)
        oh = (bt == lax.broadcasted_iota(jnp.int32, (_N, _NG), 1)
              ).astype(jnp.float32)
        ssum = lax.dot_general(oh, hs, (((0,), (0,)), ((), ())),
                               preferred_element_type=jnp.float32)
        cnt = jnp.maximum(ssum[:, _HP - 1:_HP], 1.0)
        x2 = ssum / cnt
        neg = jnp.float32(-jnp.inf)

        gidx = lax.broadcasted_iota(jnp.int32, (_NG, _HP), 0)

        def mbody(g, acc):
            row = jnp.max(jnp.where(bt == g, hn, neg), axis=0, keepdims=True)
            return jnp.where(gidx == g, row, acc)

        x1 = lax.fori_loop(0, _NG, mbody,
                           jnp.full((_NG, _HP), neg))
        lane2 = lax.broadcasted_iota(jnp.int32, (_NG, _HP), 1)
        x1 = jnp.where(lane2 < _H, x1, 0.0)
        x2 = jnp.where(lane2 < _H, x2, 0.0)
        z = (jnp.dot(x1, w1a_ref[...], preferred_element_type=jnp.float32)
             + jnp.dot(x2, w1b_ref[...], preferred_element_type=jnp.float32)
             + bn1_ref[...])
        z = jnp.maximum(z, 0.0)
        o = jnp.dot(z, w2_ref[...], preferred_element_type=jnp.float32) \
            + bn2_ref[...]
        out_ref[...] = 1.0 / (1.0 + jnp.exp(-o))

    return pl.pallas_call(
        body,
        out_shape=jax.ShapeDtypeStruct((_NG, 8), jnp.float32),
    )(parts, r2, bl2p, g2p, b2p, bt2d, W1a, W1b, bln1, W2p, bln2)


def kernel(x, edge_index, batch, Wl1, bl1, Wr1, gamma1, beta1,
           Wl2, bl2, Wr2, gamma2, beta2, Wlin1, blin1, Wlin2, blin2):
    f32 = jnp.float32
    Wl1p = jnp.zeros((_F, _HP), f32).at[:, :_H].set(Wl1)
    Wr1p = jnp.zeros((_F, _HP), f32).at[:, :_H].set(Wr1)
    Wl2p = jnp.zeros((_HP, _HP), f32).at[:_H, :_H].set(Wl2)
    Wr2p = jnp.zeros((_HP, _HP), f32).at[:_H, :_H].set(Wr2)
    bl1p = jnp.zeros((1, _HP), f32).at[0, :_H].set(bl1)
    bl2p = jnp.zeros((1, _HP), f32).at[0, :_H].set(bl2)
    g1p = jnp.zeros((1, _HP), f32).at[0, :_H].set(gamma1)
    g2p = jnp.zeros((1, _HP), f32).at[0, :_H].set(gamma2)
    b1p = jnp.zeros((1, _HP), f32).at[0, :_H].set(beta1)
    b2p = jnp.zeros((1, _HP), f32).at[0, :_H].set(beta2)
    W1a = jnp.zeros((_HP, 16), f32).at[:_H, :10].set(Wlin1[:_H])
    W1b = jnp.zeros((_HP, 16), f32).at[:_H, :10].set(Wlin1[_H:])
    bln1 = jnp.zeros((1, 16), f32).at[0, :10].set(blin1)
    W2p = jnp.zeros((16, 8), f32).at[:10, 0].set(Wlin2[:, 0])
    bln2 = jnp.zeros((1, 8), f32).at[0, 0].set(blin2[0])

    src = edge_index[0]
    dst = edge_index[1]
    pad = _EPAD - _E
    srcb = jnp.concatenate(
        [src, jnp.zeros((pad,), jnp.int32)]).reshape(_NBLKS, _BLK)
    dstb = jnp.concatenate(
        [dst, jnp.full((pad,), _N, jnp.int32)]).reshape(_NBLKS, _BLK)
    zeros = jnp.zeros((_NPAD, _HP), f32)
    bt2d = batch.reshape(_N, 1)

    y1, r1 = _tc_dense_in(x, Wl1p, Wr1p)
    parts1 = _sc_segsum(y1, srcb, dstb, zeros)
    y2, r2 = _tc_mid(parts1, r1, bl1p, g1p, b1p, Wl2p, Wr2p)
    parts2 = _sc_segsum(y2, srcb, dstb, zeros)
    out = _tc_final(parts2, r2, bl2p, g2p, b2p, bt2d, W1a, W1b, bln1, W2p, bln2)
    return out[:, :1]
